# baseline (speedup 1.0000x reference)
import functools

import jax
import jax.numpy as jnp
from jax import lax
from jax.experimental import pallas as pl
from jax.experimental.pallas import tpu as pltpu
from jax.experimental.pallas import tpu_sc as plsc

B, S, D = 1, 2048, 1024
FF = 2048
E = 8
K = 2

BLK = 256
NB = (S * K) // BLK + E
PADMAX = NB * BLK
FT2 = 512
NF2 = FF // FT2


def _routing_body(x_ref, gw_ref, gb_ref, ei_ref, pw_ref):
    g = jnp.dot(x_ref[...], gw_ref[...], preferred_element_type=jnp.float32)
    g = g + gb_ref[...]
    idx = jax.lax.broadcasted_iota(jnp.int32, (S, E), 1)
    m1 = jnp.max(g, axis=1, keepdims=True)
    i1 = jnp.min(jnp.where(g == m1, idx, E), axis=1, keepdims=True)
    gm = jnp.where(idx == i1, -1e30, g)
    m2 = jnp.max(gm, axis=1, keepdims=True)
    i2 = jnp.min(jnp.where(gm == m2, idx, E), axis=1, keepdims=True)
    z = jnp.exp(m2 - m1)
    p1 = 1.0 / (1.0 + z)
    ei_ref[...] = jnp.concatenate([i1, i2], axis=1)
    pw_ref[...] = jnp.concatenate([p1, 1.0 - p1], axis=1)


def _routing(x2d, gate_w, gate_b):
    return pl.pallas_call(
        _routing_body,
        out_shape=(
            jax.ShapeDtypeStruct((S, K), jnp.int32),
            jax.ShapeDtypeStruct((S, K), jnp.float32),
        ),
    )(x2d, gate_w, gate_b.reshape(1, E))


def _ffn_body(be_ref, nb_ref, xs_ref, w1_ref, w2_ref, wsig_ref, w3_ref,
              b1_ref, b2_ref, bsig_ref, b3_ref, wrow_ref, out_ref):
    b = pl.program_id(0)

    @pl.when(b < nb_ref[0])
    def _():
        xs = xs_ref[...]
        x1 = jnp.dot(xs, w1_ref[0], preferred_element_type=jnp.float32)
        x1 = x1 + b1_ref[0]
        x2 = jnp.dot(xs, w2_ref[0], preferred_element_type=jnp.float32)
        x2 = x2 + b2_ref[0]
        p = (x1 * x2).astype(jnp.bfloat16)
        h = jnp.dot(p, wsig_ref[0], preferred_element_type=jnp.float32)
        h = h + bsig_ref[0]
        h = (h * jax.nn.sigmoid(h)).astype(jnp.bfloat16)
        eo = jnp.dot(h, w3_ref[0],
                     preferred_element_type=jnp.float32) + b3_ref[0]
        out_ref[...] = eo * wrow_ref[0]


def _grouped_ffn(block_expert, nblocks, xs, w_sorted,
                 w1, b1, w2, b2, w3, b3, wsig, bsig):
    grid_spec = pltpu.PrefetchScalarGridSpec(
        num_scalar_prefetch=2,
        grid=(NB,),
        in_specs=[
            pl.BlockSpec((BLK, D), lambda b, be, nb: (b, 0)),
            pl.BlockSpec((1, D, FF), lambda b, be, nb: (be[b], 0, 0)),
            pl.BlockSpec((1, D, FF), lambda b, be, nb: (be[b], 0, 0)),
            pl.BlockSpec((1, FF, FF), lambda b, be, nb: (be[b], 0, 0)),
            pl.BlockSpec((1, FF, D), lambda b, be, nb: (be[b], 0, 0)),
            pl.BlockSpec((1, 1, FF), lambda b, be, nb: (be[b], 0, 0)),
            pl.BlockSpec((1, 1, FF), lambda b, be, nb: (be[b], 0, 0)),
            pl.BlockSpec((1, 1, FF), lambda b, be, nb: (be[b], 0, 0)),
            pl.BlockSpec((1, 1, D), lambda b, be, nb: (be[b], 0, 0)),
            pl.BlockSpec((1, BLK, 1), lambda b, be, nb: (b, 0, 0)),
        ],
        out_specs=pl.BlockSpec((BLK, D), lambda b, be, nb: (b, 0)),
    )
    return pl.pallas_call(
        _ffn_body,
        grid_spec=grid_spec,
        out_shape=jax.ShapeDtypeStruct((PADMAX, D), jnp.float32),
        compiler_params=pltpu.CompilerParams(
            dimension_semantics=("arbitrary",),
        ),
    )(block_expert, nblocks, xs.astype(jnp.bfloat16),
      w1.astype(jnp.bfloat16), w2.astype(jnp.bfloat16),
      wsig.astype(jnp.bfloat16), w3.astype(jnp.bfloat16),
      b1.reshape(E, 1, FF), b2.reshape(E, 1, FF),
      bsig.reshape(E, 1, FF), b3.reshape(E, 1, D),
      w_sorted.reshape(NB, BLK, 1))


NW = 32
TPW = S // NW
CH = 32


def _combine_body(rows_hbm, pos1_hbm, pos2_hbm, out_hbm,
                  idx1_v, idx2_v, r1_v, r2_v, sem1, sem2):
    wid = lax.axis_index("s") * 2 + lax.axis_index("c")
    for c in range(TPW // CH):
        base = wid * TPW + c * CH
        pltpu.sync_copy(pos1_hbm.at[pl.ds(base, CH)], idx1_v)
        pltpu.sync_copy(pos2_hbm.at[pl.ds(base, CH)], idx2_v)
        cp1 = pltpu.async_copy(rows_hbm.at[idx1_v], r1_v, sem1)
        cp2 = pltpu.async_copy(rows_hbm.at[idx2_v], r2_v, sem2)
        cp1.wait()
        cp2.wait()

        def body(i, carry):
            for j in range(D // 16):
                sl = pl.ds(j * 16, 16)
                r1_v[i, sl] = r1_v[i, sl] + r2_v[i, sl]
            return carry

        lax.fori_loop(0, CH, body, 0)
        pltpu.sync_copy(r1_v, out_hbm.at[pl.ds(base, CH)])


def _combine(rows, pos1, pos2):
    mesh = plsc.VectorSubcoreMesh(core_axis_name="c", subcore_axis_name="s")
    f = functools.partial(
        pl.kernel,
        mesh=mesh,
        out_type=jax.ShapeDtypeStruct((S, D), jnp.float32),
        scratch_types=[
            pltpu.VMEM((CH,), jnp.int32),
            pltpu.VMEM((CH,), jnp.int32),
            pltpu.VMEM((CH, D), jnp.float32),
            pltpu.VMEM((CH, D), jnp.float32),
            pltpu.SemaphoreType.DMA,
            pltpu.SemaphoreType.DMA,
        ],
    )(_combine_body)
    return f(rows, pos1, pos2)


def kernel(x, gate_w, gate_b, w1, b1, w2, b2, w3, b3, wsig, bsig):
    x2d = x.reshape(S, D)
    ei, pw = _routing(x2d, gate_w, gate_b)

    flat_e = ei.reshape(-1)
    oh = (flat_e[:, None] == jnp.arange(E)[None, :]).astype(jnp.int32)
    csum = jnp.cumsum(oh, axis=0)
    rank = jnp.sum((csum - 1) * oh, axis=1)
    counts = csum[-1]
    blocks_per_e = (counts + BLK - 1) // BLK
    blk_start = jnp.concatenate(
        [jnp.zeros((1,), jnp.int32), jnp.cumsum(blocks_per_e)[:-1]])
    nb = jnp.sum(blocks_per_e).astype(jnp.int32)
    dest = blk_start[flat_e] * BLK + rank

    tok = jnp.repeat(jnp.arange(S, dtype=jnp.int32), K)
    tok_sorted = jnp.zeros((PADMAX,), jnp.int32).at[dest].set(tok)
    w_sorted = jnp.zeros((PADMAX,), jnp.float32).at[dest].set(pw.reshape(-1))
    xs = x2d[tok_sorted]

    bidx = jnp.arange(NB, dtype=jnp.int32)
    block_expert = jnp.sum(
        (bidx[:, None] >= blk_start[None, :]).astype(jnp.int32), axis=1) - 1
    block_expert = jnp.clip(block_expert, 0, E - 1)
    last_e = block_expert[jnp.maximum(nb - 1, 0)]
    block_expert = jnp.where(bidx < nb, block_expert, last_e)

    rows = _grouped_ffn(block_expert, nb.reshape(1), xs, w_sorted,
                        w1, b1, w2, b2, w3, b3, wsig, bsig)

    pos = dest.reshape(S, K)
    out = _combine(rows, pos[:, 0], pos[:, 1])
    return out.reshape(B, S, D)


# speedup vs baseline: 1.3765x; 1.3765x over previous
import functools

import jax
import jax.numpy as jnp
from jax import lax
from jax.experimental import pallas as pl
from jax.experimental.pallas import tpu as pltpu
from jax.experimental.pallas import tpu_sc as plsc

B, S, D = 1, 2048, 1024
FF = 2048
E = 8
K = 2

BLK = 256
NB = (S * K) // BLK + E
PADMAX = NB * BLK
FT2 = 512
NF2 = FF // FT2


def _routing_body(x_ref, gw_ref, gb_ref, ei_ref, pw_ref):
    g = jnp.dot(x_ref[...], gw_ref[...], preferred_element_type=jnp.float32)
    g = g + gb_ref[...]
    idx = jax.lax.broadcasted_iota(jnp.int32, (S, E), 1)
    m1 = jnp.max(g, axis=1, keepdims=True)
    i1 = jnp.min(jnp.where(g == m1, idx, E), axis=1, keepdims=True)
    gm = jnp.where(idx == i1, -1e30, g)
    m2 = jnp.max(gm, axis=1, keepdims=True)
    i2 = jnp.min(jnp.where(gm == m2, idx, E), axis=1, keepdims=True)
    z = jnp.exp(m2 - m1)
    p1 = 1.0 / (1.0 + z)
    ei_ref[...] = jnp.concatenate([i1, i2], axis=1)
    pw_ref[...] = jnp.concatenate([p1, 1.0 - p1], axis=1)


def _routing(x2d, gate_w, gate_b):
    return pl.pallas_call(
        _routing_body,
        out_shape=(
            jax.ShapeDtypeStruct((S, K), jnp.int32),
            jax.ShapeDtypeStruct((S, K), jnp.float32),
        ),
    )(x2d, gate_w, gate_b.reshape(1, E))


def _ffn_a_body(be_ref, nb_ref, xs_ref, w1_ref, w2_ref, b1_ref, b2_ref,
                p_ref):
    b = pl.program_id(0)

    @pl.when(b < nb_ref[0])
    def _():
        xs = xs_ref[...]
        x1 = jnp.dot(xs, w1_ref[0], preferred_element_type=jnp.float32)
        x1 = x1 + b1_ref[0]
        x2 = jnp.dot(xs, w2_ref[0], preferred_element_type=jnp.float32)
        x2 = x2 + b2_ref[0]
        p_ref[...] = x1 * x2


def _ffn_b_body(be_ref, nb_ref, p_ref, wsig_ref, w3_ref, bsig_ref, b3_ref,
                wrow_ref, out_ref):
    b = pl.program_id(0)

    @pl.when(b < nb_ref[0])
    def _():
        h = jnp.dot(p_ref[...], wsig_ref[0], preferred_element_type=jnp.float32)
        h = h + bsig_ref[0]
        h = h * jax.nn.sigmoid(h)
        eo = jnp.dot(h, w3_ref[0], preferred_element_type=jnp.float32)
        out_ref[...] = (eo + b3_ref[0]) * wrow_ref[0]


def _grouped_ffn(block_expert, nblocks, xs, w_sorted,
                 w1, b1, w2, b2, w3, b3, wsig, bsig):
    spec_a = pltpu.PrefetchScalarGridSpec(
        num_scalar_prefetch=2,
        grid=(NB,),
        in_specs=[
            pl.BlockSpec((BLK, D), lambda b, be, nb: (b, 0)),
            pl.BlockSpec((1, D, FF), lambda b, be, nb: (be[b], 0, 0)),
            pl.BlockSpec((1, D, FF), lambda b, be, nb: (be[b], 0, 0)),
            pl.BlockSpec((1, 1, FF), lambda b, be, nb: (be[b], 0, 0)),
            pl.BlockSpec((1, 1, FF), lambda b, be, nb: (be[b], 0, 0)),
        ],
        out_specs=pl.BlockSpec((BLK, FF), lambda b, be, nb: (b, 0)),
    )
    p = pl.pallas_call(
        _ffn_a_body,
        grid_spec=spec_a,
        out_shape=jax.ShapeDtypeStruct((PADMAX, FF), jnp.float32),
        compiler_params=pltpu.CompilerParams(
            dimension_semantics=("arbitrary",),
        ),
    )(block_expert, nblocks, xs, w1, w2,
      b1.reshape(E, 1, FF), b2.reshape(E, 1, FF))

    spec_b = pltpu.PrefetchScalarGridSpec(
        num_scalar_prefetch=2,
        grid=(NB,),
        in_specs=[
            pl.BlockSpec((BLK, FF), lambda b, be, nb: (b, 0)),
            pl.BlockSpec((1, FF, FF), lambda b, be, nb: (be[b], 0, 0)),
            pl.BlockSpec((1, FF, D), lambda b, be, nb: (be[b], 0, 0)),
            pl.BlockSpec((1, 1, FF), lambda b, be, nb: (be[b], 0, 0)),
            pl.BlockSpec((1, 1, D), lambda b, be, nb: (be[b], 0, 0)),
            pl.BlockSpec((1, BLK, 1), lambda b, be, nb: (b, 0, 0)),
        ],
        out_specs=pl.BlockSpec((BLK, D), lambda b, be, nb: (b, 0)),
    )
    return pl.pallas_call(
        _ffn_b_body,
        grid_spec=spec_b,
        out_shape=jax.ShapeDtypeStruct((PADMAX, D), jnp.float32),
        compiler_params=pltpu.CompilerParams(
            dimension_semantics=("arbitrary",),
        ),
    )(block_expert, nblocks, p, wsig, w3,
      bsig.reshape(E, 1, FF), b3.reshape(E, 1, D),
      w_sorted.reshape(NB, BLK, 1))


NW = 32
TPW = S // NW
CH = 32


def _dispatch_body(x_hbm, d1_hbm, d2_hbm, xs_hbm,
                   idx1_v, idx2_v, rows_v, sem1, sem2):
    wid = lax.axis_index("s") * 2 + lax.axis_index("c")
    base = wid * TPW
    pltpu.sync_copy(x_hbm.at[pl.ds(base, TPW)], rows_v)
    pltpu.sync_copy(d1_hbm.at[pl.ds(base, TPW)], idx1_v)
    pltpu.sync_copy(d2_hbm.at[pl.ds(base, TPW)], idx2_v)
    cp1 = pltpu.async_copy(rows_v, xs_hbm.at[idx1_v], sem1)
    cp2 = pltpu.async_copy(rows_v, xs_hbm.at[idx2_v], sem2)
    cp1.wait()
    cp2.wait()


def _dispatch(x2d, dest1, dest2):
    mesh = plsc.VectorSubcoreMesh(core_axis_name="c", subcore_axis_name="s")
    f = functools.partial(
        pl.kernel,
        mesh=mesh,
        out_type=jax.ShapeDtypeStruct((PADMAX, D), jnp.float32),
        scratch_types=[
            pltpu.VMEM((TPW,), jnp.int32),
            pltpu.VMEM((TPW,), jnp.int32),
            pltpu.VMEM((TPW, D), jnp.float32),
            pltpu.SemaphoreType.DMA,
            pltpu.SemaphoreType.DMA,
        ],
    )(_dispatch_body)
    return f(x2d, dest1, dest2)


def _combine_body(rows_hbm, pos1_hbm, pos2_hbm, out_hbm,
                  idx1_v, idx2_v, r1_v, r2_v, sem1, sem2):
    wid = lax.axis_index("s") * 2 + lax.axis_index("c")
    for c in range(TPW // CH):
        base = wid * TPW + c * CH
        pltpu.sync_copy(pos1_hbm.at[pl.ds(base, CH)], idx1_v)
        pltpu.sync_copy(pos2_hbm.at[pl.ds(base, CH)], idx2_v)
        cp1 = pltpu.async_copy(rows_hbm.at[idx1_v], r1_v, sem1)
        cp2 = pltpu.async_copy(rows_hbm.at[idx2_v], r2_v, sem2)
        cp1.wait()
        cp2.wait()

        def body(i, carry):
            for j in range(D // 16):
                sl = pl.ds(j * 16, 16)
                r1_v[i, sl] = r1_v[i, sl] + r2_v[i, sl]
            return carry

        lax.fori_loop(0, CH, body, 0)
        pltpu.sync_copy(r1_v, out_hbm.at[pl.ds(base, CH)])


def _combine(rows, pos1, pos2):
    mesh = plsc.VectorSubcoreMesh(core_axis_name="c", subcore_axis_name="s")
    f = functools.partial(
        pl.kernel,
        mesh=mesh,
        out_type=jax.ShapeDtypeStruct((S, D), jnp.float32),
        scratch_types=[
            pltpu.VMEM((CH,), jnp.int32),
            pltpu.VMEM((CH,), jnp.int32),
            pltpu.VMEM((CH, D), jnp.float32),
            pltpu.VMEM((CH, D), jnp.float32),
            pltpu.SemaphoreType.DMA,
            pltpu.SemaphoreType.DMA,
        ],
    )(_combine_body)
    return f(rows, pos1, pos2)


def kernel(x, gate_w, gate_b, w1, b1, w2, b2, w3, b3, wsig, bsig):
    x2d = x.reshape(S, D)
    ei, pw = _routing(x2d, gate_w, gate_b)

    flat_e = ei.reshape(-1)
    oh = (flat_e[:, None] == jnp.arange(E)[None, :]).astype(jnp.int32)
    csum = jnp.cumsum(oh, axis=0)
    rank = jnp.sum((csum - 1) * oh, axis=1)
    counts = csum[-1]
    blocks_per_e = (counts + BLK - 1) // BLK
    blk_start = jnp.concatenate(
        [jnp.zeros((1,), jnp.int32), jnp.cumsum(blocks_per_e)[:-1]])
    nb = jnp.sum(blocks_per_e).astype(jnp.int32)
    dest = blk_start[flat_e] * BLK + rank

    w_sorted = jnp.zeros((PADMAX,), jnp.float32).at[dest].set(pw.reshape(-1))
    pos = dest.reshape(S, K)
    xs = _dispatch(x2d, pos[:, 0], pos[:, 1])

    bidx = jnp.arange(NB, dtype=jnp.int32)
    block_expert = jnp.sum(
        (bidx[:, None] >= blk_start[None, :]).astype(jnp.int32), axis=1) - 1
    block_expert = jnp.clip(block_expert, 0, E - 1)
    last_e = block_expert[jnp.maximum(nb - 1, 0)]
    block_expert = jnp.where(bidx < nb, block_expert, last_e)

    rows = _grouped_ffn(block_expert, nb.reshape(1), xs, w_sorted,
                        w1, b1, w2, b2, w3, b3, wsig, bsig)

    out = _combine(rows, pos[:, 0], pos[:, 1])
    return out.reshape(B, S, D)
